# baseline (speedup 1.0000x reference)
import jax
import jax.numpy as jnp
from jax.experimental import pallas as pl
import jax.experimental.pallas.tpu as pltpu

B = 2
P = 1024
DIM_IN = 32
DIM_OUT = 64
KS = 16
NN = 16
NA = 20
SIGMA = 0.08
TP = 128


def _body(xyzT_ref, xyzt_ref, fx_ref, xhl_ref, rk_ref, cka_ref, w2_ref, out_ref):
    xT = xyzT_ref[0]
    xt = xyzt_ref[0]
    FX = fx_ref[0]
    XHL = xhl_ref[0]
    rk = rk_ref[...]
    cka = cka_ref[...]
    W2p = w2_ref[...]

    d2 = None
    for d in range(3):
        diff = xT[:, d:d + 1] - xt[d:d + 1, :]
        sq = diff * diff
        d2 = sq if d2 is None else d2 + sq
    iota0 = jax.lax.broadcasted_iota(jnp.int32, (P, TP), 0)

    inv_s = jnp.float32(1.0 / SIGMA)
    two_s = jnp.float32(2.0 / SIGMA)
    ms = []
    ohs = []
    d2w = d2
    big = jnp.float32(3.0e38)
    for _ in range(NN):
        m = jnp.min(d2w, axis=0, keepdims=True)
        am = jnp.min(jnp.where(d2w == m, iota0, P), axis=0, keepdims=True)
        d2w = jnp.where(iota0 == am, big, d2w)
        ohs.append(jnp.where(iota0 == am, jnp.float32(1.0),
                             jnp.float32(0.0)).astype(jnp.bfloat16))
        ms.append(m)
    OH = jnp.concatenate(ohs, axis=1)
    d2sel = jnp.concatenate(ms, axis=1)

    gat = jnp.dot(FX, OH, preferred_element_type=jnp.float32)
    gx = jnp.dot(XHL, OH, preferred_element_type=jnp.float32)
    g = gx[0:3, :] + gx[3:6, :] - jnp.tile(xt, (1, NN))
    grk = (rk[:, 0:1] * g[0:1, :] + rk[:, 1:2] * g[1:2, :]
           + rk[:, 2:3] * g[2:3, :])
    wj = jnp.maximum(cka - d2sel * inv_s + grk * two_s, 0.0)

    outs = []
    for a in range(NA):
        fa = gat[a * DIM_IN:(a + 1) * DIM_IN, :]
        wa = wj[a * KS:(a + 1) * KS, :]
        acc = None
        for n in range(NN):
            prod = (wa[:, n * TP:(n + 1) * TP][:, None, :]
                    * fa[:, n * TP:(n + 1) * TP][None, :, :])
            acc = prod if acc is None else acc + prod
        outs.append(jnp.dot(W2p, acc.reshape(KS * DIM_IN, TP),
                            preferred_element_type=jnp.float32))
    for a in range(NA):
        out_ref[0, :, a, :] = outs[a]


def kernel(xyz, feats, anchors, kernels, W):
    b = xyz.shape[0]
    rot = jnp.transpose(jnp.matmul(anchors, kernels.T), (1, 0, 2))
    rk = jnp.transpose(rot, (1, 2, 0)).reshape(NA * KS, 3)
    cka = 1.0 - jnp.sum(rk * rk, axis=1, keepdims=True) / SIGMA

    xyzT = jnp.transpose(xyz, (0, 2, 1))
    x_hi = xyz.astype(jnp.bfloat16)
    x_lo = (xyz - x_hi.astype(jnp.float32)).astype(jnp.bfloat16)
    xhl = jnp.concatenate([x_hi, x_lo], axis=1)
    fx = jnp.transpose(feats.astype(jnp.bfloat16),
                       (0, 3, 1, 2)).reshape(b, NA * DIM_IN, P)
    W2p = jnp.transpose(W.reshape(DIM_OUT, DIM_IN, KS), (0, 2, 1)).reshape(DIM_OUT, KS * DIM_IN)

    grid = (b, P // TP)
    out = pl.pallas_call(
        _body,
        grid=grid,
        in_specs=[
            pl.BlockSpec((1, P, 3), lambda i, j: (i, 0, 0)),
            pl.BlockSpec((1, 3, TP), lambda i, j: (i, 0, j)),
            pl.BlockSpec((1, NA * DIM_IN, P), lambda i, j: (i, 0, 0)),
            pl.BlockSpec((1, 6, P), lambda i, j: (i, 0, 0)),
            pl.BlockSpec((NA * KS, 3), lambda i, j: (0, 0)),
            pl.BlockSpec((NA * KS, 1), lambda i, j: (0, 0)),
            pl.BlockSpec((DIM_OUT, DIM_IN * KS), lambda i, j: (0, 0)),
        ],
        out_specs=pl.BlockSpec((1, DIM_OUT, NA, TP), lambda i, j: (i, 0, 0, j)),
        out_shape=jax.ShapeDtypeStruct((b, DIM_OUT, NA, P), jnp.float32),
        compiler_params=pltpu.CompilerParams(
            dimension_semantics=("parallel", "parallel")),
    )(xyzT, xyz, fx, xhl, rk, cka, W2p)
    return jnp.transpose(out, (0, 1, 3, 2))


# speedup vs baseline: 1.0740x; 1.0740x over previous
import jax
import jax.numpy as jnp
from jax.experimental import pallas as pl

B = 2
P = 1024
DIM_IN = 32
DIM_OUT = 64
KS = 16
NN = 16
NA = 20
SIGMA = 0.08
TP = 128


def _body(xyzT_ref, xyzt_ref, fx_ref, rk_ref, cka_ref, w2_ref, out_ref):
    xT = xyzT_ref[0]
    xt = xyzt_ref[0]
    FX = fx_ref[0]
    rk = rk_ref[...]
    cka = cka_ref[...]
    W2p = w2_ref[...]

    d2 = None
    for d in range(3):
        diff = xT[:, d:d + 1] - xt[d:d + 1, :]
        sq = diff * diff
        d2 = sq if d2 is None else d2 + sq
    iota0 = jax.lax.broadcasted_iota(jnp.int32, (P, TP), 0)

    inv_s = jnp.float32(1.0 / SIGMA)
    big = jnp.float32(3.0e38)
    ms = []
    ohs = []
    d2w = d2
    for _ in range(NN):
        m = jnp.min(d2w, axis=0, keepdims=True)
        am = jnp.min(jnp.where(d2w == m, iota0, P), axis=0, keepdims=True)
        d2w = jnp.where(iota0 == am, big, d2w)
        ohs.append(jnp.where(iota0 == am, jnp.float32(1.0),
                             jnp.float32(0.0)).astype(jnp.bfloat16))
        ms.append(m)
    OH = jnp.concatenate(ohs, axis=1)
    mall = jnp.concatenate(ms, axis=1)

    gat = jnp.dot(FX, OH, preferred_element_type=jnp.float32)
    g = gat[640:643, :] + gat[643:646, :] - jnp.tile(xt, (1, NN))
    grk = (rk[:, 0:1] * g[0:1, :] + rk[:, 1:2] * g[1:2, :]
           + rk[:, 2:3] * g[2:3, :])
    wj = jnp.maximum((cka - mall * inv_s) + grk, 0.0)

    for a in range(NA):
        fa = gat[a * DIM_IN:(a + 1) * DIM_IN, :]
        wa = wj[a * KS:(a + 1) * KS, :]
        acc = None
        for n in range(NN):
            prod = (wa[:, n * TP:(n + 1) * TP][:, None, :]
                    * fa[:, n * TP:(n + 1) * TP][None, :, :])
            acc = prod if acc is None else acc + prod
        out_ref[0, :, a, :] = jnp.dot(W2p, acc.reshape(KS * DIM_IN, TP),
                                      preferred_element_type=jnp.float32)


def kernel(xyz, feats, anchors, kernels, W):
    b = xyz.shape[0]
    rot = jnp.transpose(jnp.matmul(anchors, kernels.T), (1, 0, 2))
    rk = jnp.transpose(rot, (1, 2, 0)).reshape(NA * KS, 3)
    cka = 1.0 - jnp.sum(rk * rk, axis=1, keepdims=True) / SIGMA
    rk = rk * (2.0 / SIGMA)

    xyzT = jnp.transpose(xyz, (0, 2, 1))
    x_hi = xyz.astype(jnp.bfloat16)
    x_lo = (xyz - x_hi.astype(jnp.float32)).astype(jnp.bfloat16)
    featsr = jnp.transpose(feats, (0, 3, 1, 2)).reshape(b, NA * DIM_IN, P)
    fx = jnp.concatenate([featsr.astype(jnp.bfloat16), x_hi, x_lo], axis=1)
    W2p = jnp.transpose(W.reshape(DIM_OUT, DIM_IN, KS), (0, 2, 1)).reshape(DIM_OUT, KS * DIM_IN)

    grid = (b, P // TP)
    out = pl.pallas_call(
        _body,
        grid=grid,
        in_specs=[
            pl.BlockSpec((1, P, 3), lambda i, j: (i, 0, 0)),
            pl.BlockSpec((1, 3, TP), lambda i, j: (i, 0, j)),
            pl.BlockSpec((1, NA * DIM_IN + 6, P), lambda i, j: (i, 0, 0)),
            pl.BlockSpec((NA * KS, 3), lambda i, j: (0, 0)),
            pl.BlockSpec((NA * KS, 1), lambda i, j: (0, 0)),
            pl.BlockSpec((DIM_OUT, DIM_IN * KS), lambda i, j: (0, 0)),
        ],
        out_specs=pl.BlockSpec((1, DIM_OUT, NA, TP), lambda i, j: (i, 0, 0, j)),
        out_shape=jax.ShapeDtypeStruct((b, DIM_OUT, NA, P), jnp.float32),
    )(xyzT, xyz, fx, rk, cka, W2p)
    return jnp.transpose(out, (0, 1, 3, 2))
